# baseline (speedup 1.0000x reference)
import jax
import jax.numpy as jnp
from jax import lax
from jax.experimental import pallas as pl
from jax.experimental.pallas import tpu as pltpu
from jax.experimental.pallas import tpu_sc as plsc

L = 4
N = 100000
D = 128
B = 512

ROWS = L * N
CHUNK = 128
SPAN = 12504
SPAN_LAST = N - 7 * SPAN
CNT = 98
NBUF = 3
ACC_ROWS = L * B


def _sc_body(hs_ref, bat_ref, out_ref, idx_dbuf, dbuf, stage, acc,
             sem_di0, sem_di1, sem_di2, sem_ld0, sem_ld1, sem_ld2,
             sem_sc0, sem_sc1, sem_sc2):
    c = lax.axis_index("c")
    s = lax.axis_index("s")
    loc_l = s // 8
    sl = s % 8
    layer = 2 * c + loc_l
    span = jnp.where(sl == 7, SPAN_LAST, SPAN)
    tile_base = layer * N + sl * SPAN
    tail_base = tile_base + span - CHUNK
    tail_mask_below = CHUNK - (span - 97 * CHUNK)

    sem_di = (sem_di0, sem_di1, sem_di2)
    sem_ld = (sem_ld0, sem_ld1, sem_ld2)
    sem_sc = (sem_sc0, sem_sc1, sem_sc2)
    dslot = tuple(dbuf.at[i] for i in range(NBUF))
    islot = tuple(idx_dbuf.at[i] for i in range(NBUF))

    def chunk_base(j):
        return jnp.where(j == CNT - 1, tail_base, tile_base + j * CHUNK)

    def loads_start(j, slot):
        base = chunk_base(j)
        pltpu.async_copy(bat_ref.at[pl.ds(base, CHUNK)], islot[slot],
                         sem_di[slot])
        pltpu.async_copy(hs_ref.at[pl.ds(base, CHUNK), :], dslot[slot],
                         sem_ld[slot])

    def loads_wait(slot):
        pltpu.make_async_copy(bat_ref.at[pl.ds(0, CHUNK)], islot[slot],
                              sem_di[slot]).wait()
        pltpu.make_async_copy(hs_ref.at[pl.ds(0, CHUNK), :], dslot[slot],
                              sem_ld[slot]).wait()

    def fix_idx(j, slot):
        mask_below = jnp.where(j == CNT - 1, tail_mask_below, 0)
        for g in range(8):
            lane = g * 16 + lax.iota(jnp.int32, 16)
            v = idx_dbuf[slot, pl.ds(g * 16, 16)] + loc_l * B
            v = jnp.where(lane < mask_below, 2 * B, v)
            idx_dbuf[slot, pl.ds(g * 16, 16)] = v

    def scat_start(slot):
        pltpu.async_copy(dslot[slot], acc.at[islot[slot]],
                         sem_sc[slot], add=True)

    def scat_wait(slot):
        pltpu.make_async_copy(dslot[slot], acc.at[islot[slot]],
                              sem_sc[slot]).wait()

    loads_start(0, 0)

    zeros16 = jnp.zeros((16,), jnp.float32)
    for r in range(32):
        for g in range(8):
            stage[r, pl.ds(g * 16, 16)] = zeros16
    for k in range(2):
        pltpu.sync_copy(stage, acc.at[pl.ds(s * 64 + k * 32, 32), :])

    @pl.when(s == 0)
    def _zero_dump():
        pltpu.sync_copy(stage.at[pl.ds(0, 1), :], acc.at[pl.ds(2 * B, 1), :])

    plsc.subcore_barrier()

    def chunk_body(j, slot):
        loads_wait(slot)
        fix_idx(j, slot)
        scat_start(slot)

    chunk_body(0, 0)
    loads_start(1, 1)
    chunk_body(1, 1)
    loads_start(2, 2)

    def triple_step(t, carry):
        for k in range(3):
            j = 3 * t + 2 + k
            slot = (2 + k) % 3
            chunk_body(j, slot)
            scat_wait(k)

            @pl.when(j < CNT - 1)
            def _nl():
                loads_start(j + 1, k)

        return carry

    lax.fori_loop(0, (CNT - 2) // 3, triple_step, 0)

    scat_wait(0)
    scat_wait(1)

    plsc.subcore_barrier()

    for k in range(2):
        pltpu.sync_copy(acc.at[pl.ds(s * 64 + k * 32, 32), :], stage)
        pltpu.sync_copy(
            stage, out_ref.at[pl.ds(c * 1024 + s * 64 + k * 32, 32), :])


_sc_segsum = pl.kernel(
    _sc_body,
    out_type=jax.ShapeDtypeStruct((ACC_ROWS, D), jnp.float32),
    mesh=plsc.VectorSubcoreMesh(core_axis_name="c", subcore_axis_name="s"),
    scratch_types=[
        pltpu.VMEM((NBUF, CHUNK), jnp.int32),
        pltpu.VMEM((NBUF, CHUNK, D), jnp.float32),
        pltpu.VMEM((32, D), jnp.float32),
        pltpu.VMEM_SHARED((2 * B + 1, D), jnp.float32),
        pltpu.SemaphoreType.DMA,
        pltpu.SemaphoreType.DMA,
        pltpu.SemaphoreType.DMA,
        pltpu.SemaphoreType.DMA,
        pltpu.SemaphoreType.DMA,
        pltpu.SemaphoreType.DMA,
        pltpu.SemaphoreType.DMA,
        pltpu.SemaphoreType.DMA,
        pltpu.SemaphoreType.DMA,
    ],
)


def _proj_body(x_ref, w_ref, b_ref, o_ref):
    w = w_ref[...]
    r = jnp.broadcast_to(b_ref[...], (B, D))
    for l in range(L):
        x = x_ref[pl.ds(l * B, B), :]
        wl = w[:, l * D:(l + 1) * D]
        r = r + lax.dot_general(x, wl, (((1,), (1,)), ((), ())),
                                preferred_element_type=jnp.float32)
    o_ref[...] = r


def _project(parts, W, b2):
    return pl.pallas_call(
        _proj_body,
        out_shape=jax.ShapeDtypeStruct((B, D), jnp.float32),
    )(parts, W, b2)


@jax.jit
def kernel(hs, batches, W, b):
    hs2 = hs.reshape(ROWS, D)
    bat2 = batches.reshape(ROWS).astype(jnp.int32)
    parts = _sc_segsum(hs2, bat2)
    return _project(parts, W, b.reshape(1, D))


# speedup vs baseline: 1.0841x; 1.0841x over previous
import jax
import jax.numpy as jnp
from jax import lax
from jax.experimental import pallas as pl
from jax.experimental.pallas import tpu as pltpu
from jax.experimental.pallas import tpu_sc as plsc

L = 4
N = 100000
D = 128
B = 512

ROWS = L * N
CHUNK = 128
SPAN = 12504
SPAN_LAST = N - 7 * SPAN
CNT = 98
NBUF = 7
LA = 4
ACC_ROWS = L * B


def _sc_body(hs_ref, bat_ref, out_ref, idx_dbuf, dbuf, stage, acc,
             sem_l0, sem_l1, sem_l2, sem_l3, sem_l4, sem_l5, sem_l6,
             sem_s0, sem_s1, sem_s2, sem_s3, sem_s4, sem_s5, sem_s6):
    c = lax.axis_index("c")
    s = lax.axis_index("s")
    loc_l = s // 8
    sl = s % 8
    layer = 2 * c + loc_l
    span = jnp.where(sl == 7, SPAN_LAST, SPAN)
    tile_base = layer * N + sl * SPAN
    tail_base = tile_base + span - CHUNK
    tail_mask_below = CHUNK - (span - 97 * CHUNK)

    sem_ld = (sem_l0, sem_l1, sem_l2, sem_l3, sem_l4, sem_l5, sem_l6)
    sem_sc = (sem_s0, sem_s1, sem_s2, sem_s3, sem_s4, sem_s5, sem_s6)
    dslot = tuple(dbuf.at[i] for i in range(NBUF))
    islot = tuple(idx_dbuf.at[i] for i in range(NBUF))

    def chunk_base(j):
        return jnp.where(j == CNT - 1, tail_base, tile_base + j * CHUNK)

    def loads_start(j, slot):
        base = chunk_base(j)
        pltpu.async_copy(bat_ref.at[pl.ds(base, CHUNK)], islot[slot],
                         sem_ld[slot])
        pltpu.async_copy(hs_ref.at[pl.ds(base, CHUNK), :], dslot[slot],
                         sem_ld[slot])

    def loads_wait(slot):
        pltpu.make_async_copy(bat_ref.at[pl.ds(0, CHUNK)], islot[slot],
                              sem_ld[slot]).wait()
        pltpu.make_async_copy(hs_ref.at[pl.ds(0, CHUNK), :], dslot[slot],
                              sem_ld[slot]).wait()

    def fix_idx(j, slot):
        mask_below = jnp.where(j == CNT - 1, tail_mask_below, 0)
        for g in range(8):
            lane = g * 16 + lax.iota(jnp.int32, 16)
            v = idx_dbuf[slot, pl.ds(g * 16, 16)] + loc_l * B
            v = jnp.where(lane < mask_below, 2 * B, v)
            idx_dbuf[slot, pl.ds(g * 16, 16)] = v

    def scat_start(slot):
        pltpu.async_copy(dslot[slot], acc.at[islot[slot]],
                         sem_sc[slot], add=True)

    def scat_wait(slot):
        pltpu.make_async_copy(dslot[slot], acc.at[islot[slot]],
                              sem_sc[slot]).wait()

    for j in range(LA):
        loads_start(j, j)

    zeros16 = jnp.zeros((16,), jnp.float32)
    for r in range(32):
        for g in range(8):
            stage[r, pl.ds(g * 16, 16)] = zeros16
    for k in range(2):
        pltpu.sync_copy(stage, acc.at[pl.ds(s * 64 + k * 32, 32), :])

    @pl.when(s == 0)
    def _zero_dump():
        pltpu.sync_copy(stage.at[pl.ds(0, 1), :], acc.at[pl.ds(2 * B, 1), :])

    plsc.subcore_barrier()

    def septet_step(t, carry):
        for k in range(7):
            j = 7 * t + k
            loads_wait(k)
            fix_idx(j, k)
            scat_start(k)

            @pl.when(j >= 7 - LA)
            def _ws():
                scat_wait((k + LA) % 7)

            @pl.when(j + LA < CNT)
            def _nl():
                loads_start(j + LA, (k + LA) % 7)

        return carry

    lax.fori_loop(0, CNT // 7, septet_step, 0)

    for j in range(CNT - (7 - LA), CNT):
        scat_wait(j % 7)

    plsc.subcore_barrier()

    for k in range(2):
        pltpu.sync_copy(acc.at[pl.ds(s * 64 + k * 32, 32), :], stage)
        pltpu.sync_copy(
            stage, out_ref.at[pl.ds(c * 1024 + s * 64 + k * 32, 32), :])


_sc_segsum = pl.kernel(
    _sc_body,
    out_type=jax.ShapeDtypeStruct((ACC_ROWS, D), jnp.float32),
    mesh=plsc.VectorSubcoreMesh(core_axis_name="c", subcore_axis_name="s"),
    scratch_types=[
        pltpu.VMEM((NBUF, CHUNK), jnp.int32),
        pltpu.VMEM((NBUF, CHUNK, D), jnp.float32),
        pltpu.VMEM((32, D), jnp.float32),
        pltpu.VMEM_SHARED((2 * B + 1, D), jnp.float32),
    ] + [pltpu.SemaphoreType.DMA] * 14,
)


def _proj_body(x_ref, w_ref, b_ref, o_ref):
    w = w_ref[...]
    r = jnp.broadcast_to(b_ref[...], (B, D))
    for l in range(L):
        x = x_ref[pl.ds(l * B, B), :]
        wl = w[:, l * D:(l + 1) * D]
        r = r + lax.dot_general(x, wl, (((1,), (1,)), ((), ())),
                                preferred_element_type=jnp.float32)
    o_ref[...] = r


def _project(parts, W, b2):
    return pl.pallas_call(
        _proj_body,
        out_shape=jax.ShapeDtypeStruct((B, D), jnp.float32),
    )(parts, W, b2)


@jax.jit
def kernel(hs, batches, W, b):
    hs2 = hs.reshape(ROWS, D)
    bat2 = batches.reshape(ROWS).astype(jnp.int32)
    parts = _sc_segsum(hs2, bat2)
    return _project(parts, W, b.reshape(1, D))
